# Initial kernel scaffold; baseline (speedup 1.0000x reference)
import jax
import jax.numpy as jnp
from jax.experimental import pallas as pl


def kernel(features, edges, gcn_w0, gcn_b0, gcn_w1, gcn_b1, pc_w, pc_b, hc_w):
    raise NotImplementedError("write your pallas kernel here")


# speedup vs baseline: 14.4378x; 14.4378x over previous
import functools

import jax
import jax.numpy as jnp
from jax import lax
from jax.experimental import pallas as pl
from jax.experimental.pallas import tpu as pltpu
from jax.experimental.pallas import tpu_sc as plsc

N = 10000
NPAD = 10240
F = 128
FH = 64
E = 320000
K = 128
CH = 157
EPAD = 16 * CH * K
NT = 16
NC = 2
RB = 640
NUM_CAPS = 10
NUM_TARGETS = 10
CAP_DIM = 8
B_CAPS = 1000

_SC_MESH = dict(core_axis_name="c", subcore_axis_name="s")


_DW = 16
_NCH0 = 79


def _deg_body(dst_hbm, out_hbm, dstv, buf, acc):
    c = lax.axis_index("c")
    s = lax.axis_index("s")
    pltpu.sync_copy(dst_hbm.at[s], dstv)

    zero16 = jnp.zeros((16,), jnp.float32)

    def zb(q, carry):
        buf[q] = zero16
        return carry

    lax.fori_loop(0, K, zb, 0)
    base = s * RB

    def ib(q, carry):
        pltpu.sync_copy(buf, acc.at[pl.ds(base + q * K, K)])
        return carry

    lax.fori_loop(0, RB // K, ib, 0)
    plsc.subcore_barrier()

    one16 = jnp.ones((16,), jnp.float32)

    def ob(q, carry):
        buf[q] = one16
        return carry

    lax.fori_loop(0, K, ob, 0)
    start = c * _NCH0

    def eb(j, carry):
        pltpu.sync_copy(buf, acc.at[dstv.at[j]], add=True)
        return carry

    lax.fori_loop(start, start + _NCH0 - c, eb, 0)
    plsc.subcore_barrier()

    def wb(q, carry):
        pltpu.sync_copy(acc.at[pl.ds(base + q * K, K)], buf)
        pltpu.sync_copy(buf, out_hbm.at[c].at[pl.ds(base + q * K, K)])
        return carry

    lax.fori_loop(0, RB // K, wb, 0)


@functools.lru_cache(maxsize=None)
def _build_deg_kernel():
    return pl.kernel(
        _deg_body,
        out_type=jax.ShapeDtypeStruct((NC, NPAD, _DW), jnp.float32),
        mesh=plsc.VectorSubcoreMesh(**_SC_MESH),
        scratch_types=[
            pltpu.VMEM((CH, K), jnp.int32),
            pltpu.VMEM((K, _DW), jnp.float32),
            pltpu.VMEM_SHARED((NPAD, _DW), jnp.float32),
        ],
    )


def _deg_call(dst3):
    return _build_deg_kernel()(dst3)


def _agg_body(hp_hbm, src_hbm, dst_hbm, out_hbm, srcv, dstv, buf, acc, sem):
    c = lax.axis_index("c")
    s = lax.axis_index("s")
    pltpu.sync_copy(src_hbm.at[s], srcv)
    pltpu.sync_copy(dst_hbm.at[s], dstv)

    base = s * RB

    def ibody(q, carry):
        pltpu.sync_copy(hp_hbm.at[c].at[pl.ds(base + q * K, K)], buf)
        pltpu.sync_copy(buf, acc.at[pl.ds(base + q * K, K)])
        return carry

    lax.fori_loop(0, RB // K, ibody, 0)
    plsc.subcore_barrier()

    def ebody(j, carry):
        pltpu.async_copy(hp_hbm.at[c].at[srcv.at[j]], buf, sem).wait()
        pltpu.sync_copy(buf, acc.at[dstv.at[j]], add=True)
        return carry

    lax.fori_loop(0, CH, ebody, 0)
    plsc.subcore_barrier()

    def obody(q, carry):
        pltpu.sync_copy(acc.at[pl.ds(base + q * K, K)], buf)
        pltpu.sync_copy(buf, out_hbm.at[c].at[pl.ds(base + q * K, K)])
        return carry

    lax.fori_loop(0, RB // K, obody, 0)


@functools.lru_cache(maxsize=None)
def _build_agg_kernel():
    return pl.kernel(
        _agg_body,
        out_type=jax.ShapeDtypeStruct((NC, NPAD, FH), jnp.float32),
        mesh=plsc.VectorSubcoreMesh(**_SC_MESH),
        scratch_types=[
            pltpu.VMEM((CH, K), jnp.int32),
            pltpu.VMEM((CH, K), jnp.int32),
            pltpu.VMEM((K, FH), jnp.float32),
            pltpu.VMEM_SHARED((NPAD, FH), jnp.float32),
            pltpu.SemaphoreType.DMA,
        ],
        compiler_params=pltpu.CompilerParams(use_tc_tiling_on_sc=False),
    )


def _agg_call(hp, src3, dst3):
    return _build_agg_kernel()(hp, src3, dst3)


def _dinv_of(pt_blk):
    return lax.rsqrt(1.0 + pt_blk)


def _k1_body(x_ref, w_ref, pt_ref, o_ref):
    dinv = _dinv_of(pt_ref[...])
    h = jnp.dot(x_ref[...], w_ref[0], preferred_element_type=jnp.float32)
    o_ref[0] = dinv * h


def _k1(xpad, w0h, cnt):
    return pl.pallas_call(
        _k1_body,
        grid=(NPAD // RB, NC),
        in_specs=[
            pl.BlockSpec((RB, F), lambda r, c: (r, 0)),
            pl.BlockSpec((1, F, FH), lambda r, c: (c, 0, 0)),
            pl.BlockSpec((RB, 1), lambda r, c: (r, 0)),
        ],
        out_specs=pl.BlockSpec((1, RB, FH), lambda r, c: (c, r, 0)),
        out_shape=jax.ShapeDtypeStruct((NC, NPAD, FH), jnp.float32),
    )(xpad, w0h, cnt)


def _k2_body(a_ref, pt_ref, b_ref, w_ref, o_ref):
    r = pl.program_id(0)
    a = jnp.concatenate([a_ref[0], a_ref[1]], axis=1)
    dinv = _dinv_of(pt_ref[...])
    h = jnp.maximum(dinv * a + b_ref[...], 0.0)
    row = r * RB + lax.broadcasted_iota(jnp.int32, (RB, 1), 0)
    h = jnp.where(row < N, h, 0.0)
    o_ref[0] = dinv * jnp.dot(h, w_ref[0], preferred_element_type=jnp.float32)


def _k2(agg0, cnt, b0, w1h):
    return pl.pallas_call(
        _k2_body,
        grid=(NPAD // RB, NC),
        in_specs=[
            pl.BlockSpec((NC, RB, FH), lambda r, c: (0, r, 0)),
            pl.BlockSpec((RB, 1), lambda r, c: (r, 0)),
            pl.BlockSpec((1, F), lambda r, c: (0, 0)),
            pl.BlockSpec((1, F, FH), lambda r, c: (c, 0, 0)),
        ],
        out_specs=pl.BlockSpec((1, RB, FH), lambda r, c: (c, r, 0)),
        out_shape=jax.ShapeDtypeStruct((NC, NPAD, FH), jnp.float32),
    )(agg0, cnt, b0, w1h)


_R3 = 2000


def _k3_body(a_ref, pt_ref, b_ref, o_ref):
    a = jnp.concatenate([a_ref[0], a_ref[1]], axis=1)
    dinv = _dinv_of(pt_ref[...])
    o_ref[...] = jnp.maximum(dinv * a + b_ref[...], 0.0)


def _k3(agg1, cnt, b1):
    return pl.pallas_call(
        _k3_body,
        grid=(N // _R3,),
        in_specs=[
            pl.BlockSpec((NC, _R3, FH), lambda r: (0, r, 0)),
            pl.BlockSpec((_R3, 1), lambda r: (r, 0)),
            pl.BlockSpec((1, F), lambda r: (0, 0)),
        ],
        out_specs=pl.BlockSpec((_R3, F), lambda r: (r, 0)),
        out_shape=jax.ShapeDtypeStruct((N, F), jnp.float32),
    )(agg1, cnt, b1)


def _k3b_body(hid_ref, pw_ref, pb_ref, u_ref, q_ref):
    u = jnp.dot(pw_ref[...], hid_ref[...], preferred_element_type=jnp.float32)
    u = u + pb_ref[...]
    u_ref[...] = u
    q_ref[...] = jnp.sum(u * u).reshape(1, 1)


def _k3b(hidden2, pcw8, pcb):
    return pl.pallas_call(
        _k3b_body,
        out_shape=[
            jax.ShapeDtypeStruct((CAP_DIM, N), jnp.float32),
            jax.ShapeDtypeStruct((1, 1), jnp.float32),
        ],
    )(hidden2, pcw8, pcb)


def _k4_body(v_ref, q_ref, w_ref, o_ref):
    q = q_ref[...]
    scale = (q / (1.0 + q)) * lax.rsqrt(q + 1e-12)
    X = v_ref[...] * scale

    riota = lax.broadcasted_iota(jnp.int32, (80, CAP_DIM), 0)
    kiota = lax.broadcasted_iota(jnp.int32, (80, CAP_DIM), 1)
    rrow = lax.broadcasted_iota(jnp.int32, (NUM_TARGETS, 80), 0)
    rcol = lax.broadcasted_iota(jnp.int32, (NUM_TARGETS, 80), 1) // CAP_DIM
    Rm = jnp.where(rrow == rcol, 1.0, 0.0)
    scol = lax.broadcasted_iota(jnp.int32, (80, NUM_TARGETS), 0) // CAP_DIM
    sj = lax.broadcasted_iota(jnp.int32, (80, NUM_TARGETS), 1)
    Sm = jnp.where(scol == sj, 1.0, 0.0)

    uhat = []
    for i in range(NUM_CAPS):
        Pm = jnp.where((riota % NUM_CAPS == i) & (riota // NUM_CAPS == kiota),
                       1.0, 0.0)
        xi = jnp.dot(X, Pm, preferred_element_type=jnp.float32)
        uhat.append(jnp.dot(xi, w_ref[i], preferred_element_type=jnp.float32))

    brows = [jnp.zeros((1, NUM_TARGETS), jnp.float32) for _ in range(NUM_CAPS)]
    v80 = jnp.zeros((B_CAPS, 80), jnp.float32)
    for it in range(3):
        s80 = jnp.zeros((B_CAPS, 80), jnp.float32)
        for i in range(NUM_CAPS):
            b = brows[i]
            m = jnp.max(b, axis=1, keepdims=True)
            e = jnp.exp(b - m)
            ci = e / jnp.sum(e, axis=1, keepdims=True)
            cb = jnp.dot(ci, Rm, preferred_element_type=jnp.float32)
            s80 = s80 + uhat[i] * cb
        m10 = jnp.dot(s80 * s80, Sm, preferred_element_type=jnp.float32)
        sc10 = (m10 / (1.0 + m10)) * lax.rsqrt(m10 + 1e-12)
        v80 = s80 * jnp.dot(sc10, Rm, preferred_element_type=jnp.float32)
        if it < 2:
            for i in range(NUM_CAPS):
                agr = jnp.dot(uhat[i] * v80, Sm,
                              preferred_element_type=jnp.float32)
                brows[i] = brows[i] + jnp.sum(agr, axis=0, keepdims=True) * (
                    1.0 / B_CAPS)

    o_ref[...] = jnp.sum(v80, axis=0, keepdims=True) * (1.0 / B_CAPS)


def _k4(v80, ssq, hc_m):
    return pl.pallas_call(
        _k4_body,
        out_shape=jax.ShapeDtypeStruct((1, NUM_TARGETS * CAP_DIM), jnp.float32),
    )(v80, ssq, hc_m)


def kernel(features, edges, gcn_w0, gcn_b0, gcn_w1, gcn_b1, pc_w, pc_b, hc_w):
    edges = edges.astype(jnp.int32)
    src = edges[0]
    dst = edges[1]
    pad = jnp.full((EPAD - E,), N, jnp.int32)
    src3 = jnp.concatenate([src, pad]).reshape(NT, CH, K)
    dst3 = jnp.concatenate([dst, pad]).reshape(NT, CH, K)

    parts = _deg_call(dst3)
    cnt = parts[0, :, 0:1] + parts[1, :, 0:1]

    xpad = jnp.pad(features, ((0, NPAD - N), (0, 0)))
    w0h = gcn_w0.reshape(F, NC, FH).transpose(1, 0, 2)
    w1h = gcn_w1.reshape(F, NC, FH).transpose(1, 0, 2)
    hp0 = _k1(xpad, w0h, cnt)
    agg0 = _agg_call(hp0, src3, dst3)
    hp1 = _k2(agg0, cnt, gcn_b0.reshape(1, F), w1h)
    agg1 = _agg_call(hp1, src3, dst3)
    h2 = _k3(agg1, cnt, gcn_b1.reshape(1, F))

    hidden2 = h2.reshape(F, N)
    pcw8 = pc_w[:, 0, :, 0]
    u8, ssq = _k3b(hidden2, pcw8, pc_b.reshape(CAP_DIM, 1))

    v80 = u8.reshape(B_CAPS, NUM_CAPS * CAP_DIM)
    hc_m = hc_w.transpose(0, 2, 1, 3).reshape(NUM_CAPS, CAP_DIM,
                                              NUM_TARGETS * CAP_DIM)
    out80 = _k4(v80, ssq, hc_m)
    return out80.reshape(1, NUM_TARGETS, CAP_DIM)


# speedup vs baseline: 17.0587x; 1.1815x over previous
import functools

import jax
import jax.numpy as jnp
from jax import lax
from jax.experimental import pallas as pl
from jax.experimental.pallas import tpu as pltpu
from jax.experimental.pallas import tpu_sc as plsc

N = 10000
NPAD = 10240
F = 128
FH = 64
E = 320000
K = 128
CH = 157
EPAD = 16 * CH * K
NT = 16
NC = 2
RB = 640
NUM_CAPS = 10
NUM_TARGETS = 10
CAP_DIM = 8
B_CAPS = 1000

_SC_MESH = dict(core_axis_name="c", subcore_axis_name="s")


_DW = 16
_NCH0 = 79


def _deg_body(dst_hbm, out_hbm, dstv, buf, acc):
    c = lax.axis_index("c")
    s = lax.axis_index("s")
    pltpu.sync_copy(dst_hbm.at[s], dstv)

    zero16 = jnp.zeros((16,), jnp.float32)

    def zb(q, carry):
        buf[q] = zero16
        return carry

    lax.fori_loop(0, K, zb, 0)
    base = s * RB

    def ib(q, carry):
        pltpu.sync_copy(buf, acc.at[pl.ds(base + q * K, K)])
        return carry

    lax.fori_loop(0, RB // K, ib, 0)
    plsc.subcore_barrier()

    one16 = jnp.ones((16,), jnp.float32)

    def ob(q, carry):
        buf[q] = one16
        return carry

    lax.fori_loop(0, K, ob, 0)
    start = c * _NCH0

    def eb(j, carry):
        pltpu.sync_copy(buf, acc.at[dstv.at[j]], add=True)
        return carry

    lax.fori_loop(start, start + _NCH0 - c, eb, 0)
    plsc.subcore_barrier()

    def wb(q, carry):
        pltpu.sync_copy(acc.at[pl.ds(base + q * K, K)], buf)
        pltpu.sync_copy(buf, out_hbm.at[c].at[pl.ds(base + q * K, K)])
        return carry

    lax.fori_loop(0, RB // K, wb, 0)


@functools.lru_cache(maxsize=None)
def _build_deg_kernel():
    return pl.kernel(
        _deg_body,
        out_type=jax.ShapeDtypeStruct((NC, NPAD, _DW), jnp.float32),
        mesh=plsc.VectorSubcoreMesh(**_SC_MESH),
        scratch_types=[
            pltpu.VMEM((CH, K), jnp.int32),
            pltpu.VMEM((K, _DW), jnp.float32),
            pltpu.VMEM_SHARED((NPAD, _DW), jnp.float32),
        ],
    )


def _deg_call(dst3):
    return _build_deg_kernel()(dst3)


def _agg_body(hp_hbm, src_hbm, dst_hbm, out_hbm, srcv, dstv, bufa, bufb, acc,
              gsa, gsb):
    c = lax.axis_index("c")
    s = lax.axis_index("s")
    pltpu.sync_copy(src_hbm.at[s], srcv)
    pltpu.sync_copy(dst_hbm.at[s], dstv)

    base = s * RB

    def ibody(q, carry):
        pltpu.sync_copy(hp_hbm.at[c].at[pl.ds(base + q * K, K)], bufa)
        pltpu.sync_copy(bufa, acc.at[pl.ds(base + q * K, K)])
        return carry

    lax.fori_loop(0, RB // K, ibody, 0)
    plsc.subcore_barrier()

    pltpu.async_copy(hp_hbm.at[c].at[srcv.at[0]], bufa, gsa)

    def pair(p, carry):
        j0 = 2 * p
        j1 = j0 + 1
        pltpu.make_async_copy(hp_hbm.at[c].at[srcv.at[j0]], bufa, gsa).wait()
        pltpu.async_copy(hp_hbm.at[c].at[srcv.at[j1]], bufb, gsb)
        pltpu.sync_copy(bufa, acc.at[dstv.at[j0]], add=True)
        pltpu.make_async_copy(hp_hbm.at[c].at[srcv.at[j1]], bufb, gsb).wait()

        @pl.when(j0 + 2 < CH)
        def _():
            pltpu.async_copy(hp_hbm.at[c].at[srcv.at[j0 + 2]], bufa, gsa)

        pltpu.sync_copy(bufb, acc.at[dstv.at[j1]], add=True)
        return carry

    lax.fori_loop(0, CH // 2, pair, 0)
    pltpu.make_async_copy(hp_hbm.at[c].at[srcv.at[CH - 1]], bufa, gsa).wait()
    pltpu.sync_copy(bufa, acc.at[dstv.at[CH - 1]], add=True)

    plsc.subcore_barrier()

    def obody(q, carry):
        pltpu.sync_copy(acc.at[pl.ds(base + q * K, K)], bufa)
        pltpu.sync_copy(bufa, out_hbm.at[c].at[pl.ds(base + q * K, K)])
        return carry

    lax.fori_loop(0, RB // K, obody, 0)


@functools.lru_cache(maxsize=None)
def _build_agg_kernel():
    return pl.kernel(
        _agg_body,
        out_type=jax.ShapeDtypeStruct((NC, NPAD, FH), jnp.float32),
        mesh=plsc.VectorSubcoreMesh(**_SC_MESH),
        scratch_types=[
            pltpu.VMEM((CH, K), jnp.int32),
            pltpu.VMEM((CH, K), jnp.int32),
            pltpu.VMEM((K, FH), jnp.float32),
            pltpu.VMEM((K, FH), jnp.float32),
            pltpu.VMEM_SHARED((NPAD, FH), jnp.float32),
            pltpu.SemaphoreType.DMA,
            pltpu.SemaphoreType.DMA,
        ],
        compiler_params=pltpu.CompilerParams(use_tc_tiling_on_sc=False),
    )


def _agg_call(hp, src3, dst3):
    return _build_agg_kernel()(hp, src3, dst3)


def _dinv_of(pt_blk):
    return lax.rsqrt(1.0 + pt_blk)


def _k1_body(x_ref, w_ref, pt_ref, o_ref):
    dinv = _dinv_of(pt_ref[...])
    h = jnp.dot(x_ref[...], w_ref[0], preferred_element_type=jnp.float32)
    o_ref[0] = dinv * h


def _k1(xpad, w0h, cnt):
    return pl.pallas_call(
        _k1_body,
        grid=(NPAD // RB, NC),
        in_specs=[
            pl.BlockSpec((RB, F), lambda r, c: (r, 0)),
            pl.BlockSpec((1, F, FH), lambda r, c: (c, 0, 0)),
            pl.BlockSpec((RB, 1), lambda r, c: (r, 0)),
        ],
        out_specs=pl.BlockSpec((1, RB, FH), lambda r, c: (c, r, 0)),
        out_shape=jax.ShapeDtypeStruct((NC, NPAD, FH), jnp.float32),
    )(xpad, w0h, cnt)


def _k2_body(a_ref, pt_ref, b_ref, w_ref, o_ref):
    r = pl.program_id(0)
    a = jnp.concatenate([a_ref[0], a_ref[1]], axis=1)
    dinv = _dinv_of(pt_ref[...])
    h = jnp.maximum(dinv * a + b_ref[...], 0.0)
    row = r * RB + lax.broadcasted_iota(jnp.int32, (RB, 1), 0)
    h = jnp.where(row < N, h, 0.0)
    o_ref[0] = dinv * jnp.dot(h, w_ref[0], preferred_element_type=jnp.float32)


def _k2(agg0, cnt, b0, w1h):
    return pl.pallas_call(
        _k2_body,
        grid=(NPAD // RB, NC),
        in_specs=[
            pl.BlockSpec((NC, RB, FH), lambda r, c: (0, r, 0)),
            pl.BlockSpec((RB, 1), lambda r, c: (r, 0)),
            pl.BlockSpec((1, F), lambda r, c: (0, 0)),
            pl.BlockSpec((1, F, FH), lambda r, c: (c, 0, 0)),
        ],
        out_specs=pl.BlockSpec((1, RB, FH), lambda r, c: (c, r, 0)),
        out_shape=jax.ShapeDtypeStruct((NC, NPAD, FH), jnp.float32),
    )(agg0, cnt, b0, w1h)


_R3 = 2000


def _k3_body(a_ref, pt_ref, b_ref, o_ref):
    a = jnp.concatenate([a_ref[0], a_ref[1]], axis=1)
    dinv = _dinv_of(pt_ref[...])
    o_ref[...] = jnp.maximum(dinv * a + b_ref[...], 0.0)


def _k3(agg1, cnt, b1):
    return pl.pallas_call(
        _k3_body,
        grid=(N // _R3,),
        in_specs=[
            pl.BlockSpec((NC, _R3, FH), lambda r: (0, r, 0)),
            pl.BlockSpec((_R3, 1), lambda r: (r, 0)),
            pl.BlockSpec((1, F), lambda r: (0, 0)),
        ],
        out_specs=pl.BlockSpec((_R3, F), lambda r: (r, 0)),
        out_shape=jax.ShapeDtypeStruct((N, F), jnp.float32),
    )(agg1, cnt, b1)


def _k3b_body(hid_ref, pw_ref, pb_ref, u_ref, q_ref):
    u = jnp.dot(pw_ref[...], hid_ref[...], preferred_element_type=jnp.float32)
    u = u + pb_ref[...]
    u_ref[...] = u
    q_ref[...] = jnp.sum(u * u).reshape(1, 1)


def _k3b(hidden2, pcw8, pcb):
    return pl.pallas_call(
        _k3b_body,
        out_shape=[
            jax.ShapeDtypeStruct((CAP_DIM, N), jnp.float32),
            jax.ShapeDtypeStruct((1, 1), jnp.float32),
        ],
    )(hidden2, pcw8, pcb)


def _k4_body(v_ref, q_ref, w_ref, o_ref):
    q = q_ref[...]
    scale = (q / (1.0 + q)) * lax.rsqrt(q + 1e-12)
    X = v_ref[...] * scale

    riota = lax.broadcasted_iota(jnp.int32, (80, CAP_DIM), 0)
    kiota = lax.broadcasted_iota(jnp.int32, (80, CAP_DIM), 1)
    rrow = lax.broadcasted_iota(jnp.int32, (NUM_TARGETS, 80), 0)
    rcol = lax.broadcasted_iota(jnp.int32, (NUM_TARGETS, 80), 1) // CAP_DIM
    Rm = jnp.where(rrow == rcol, 1.0, 0.0)
    scol = lax.broadcasted_iota(jnp.int32, (80, NUM_TARGETS), 0) // CAP_DIM
    sj = lax.broadcasted_iota(jnp.int32, (80, NUM_TARGETS), 1)
    Sm = jnp.where(scol == sj, 1.0, 0.0)

    uhat = []
    for i in range(NUM_CAPS):
        Pm = jnp.where((riota % NUM_CAPS == i) & (riota // NUM_CAPS == kiota),
                       1.0, 0.0)
        xi = jnp.dot(X, Pm, preferred_element_type=jnp.float32)
        uhat.append(jnp.dot(xi, w_ref[i], preferred_element_type=jnp.float32))

    brows = [jnp.zeros((1, NUM_TARGETS), jnp.float32) for _ in range(NUM_CAPS)]
    v80 = jnp.zeros((B_CAPS, 80), jnp.float32)
    for it in range(3):
        s80 = jnp.zeros((B_CAPS, 80), jnp.float32)
        for i in range(NUM_CAPS):
            b = brows[i]
            m = jnp.max(b, axis=1, keepdims=True)
            e = jnp.exp(b - m)
            ci = e / jnp.sum(e, axis=1, keepdims=True)
            cb = jnp.dot(ci, Rm, preferred_element_type=jnp.float32)
            s80 = s80 + uhat[i] * cb
        m10 = jnp.dot(s80 * s80, Sm, preferred_element_type=jnp.float32)
        sc10 = (m10 / (1.0 + m10)) * lax.rsqrt(m10 + 1e-12)
        v80 = s80 * jnp.dot(sc10, Rm, preferred_element_type=jnp.float32)
        if it < 2:
            for i in range(NUM_CAPS):
                agr = jnp.dot(uhat[i] * v80, Sm,
                              preferred_element_type=jnp.float32)
                brows[i] = brows[i] + jnp.sum(agr, axis=0, keepdims=True) * (
                    1.0 / B_CAPS)

    o_ref[...] = jnp.sum(v80, axis=0, keepdims=True) * (1.0 / B_CAPS)


def _k4(v80, ssq, hc_m):
    return pl.pallas_call(
        _k4_body,
        out_shape=jax.ShapeDtypeStruct((1, NUM_TARGETS * CAP_DIM), jnp.float32),
    )(v80, ssq, hc_m)


def kernel(features, edges, gcn_w0, gcn_b0, gcn_w1, gcn_b1, pc_w, pc_b, hc_w):
    edges = edges.astype(jnp.int32)
    src = edges[0]
    dst = edges[1]
    pad = jnp.full((EPAD - E,), N, jnp.int32)
    src3 = jnp.concatenate([src, pad]).reshape(NT, CH, K)
    dst3 = jnp.concatenate([dst, pad]).reshape(NT, CH, K)

    parts = _deg_call(dst3)
    cnt = parts[0, :, 0:1] + parts[1, :, 0:1]

    xpad = jnp.pad(features, ((0, NPAD - N), (0, 0)))
    w0h = gcn_w0.reshape(F, NC, FH).transpose(1, 0, 2)
    w1h = gcn_w1.reshape(F, NC, FH).transpose(1, 0, 2)
    hp0 = _k1(xpad, w0h, cnt)
    agg0 = _agg_call(hp0, src3, dst3)
    hp1 = _k2(agg0, cnt, gcn_b0.reshape(1, F), w1h)
    agg1 = _agg_call(hp1, src3, dst3)
    h2 = _k3(agg1, cnt, gcn_b1.reshape(1, F))

    hidden2 = h2.reshape(F, N)
    pcw8 = pc_w[:, 0, :, 0]
    u8, ssq = _k3b(hidden2, pcw8, pc_b.reshape(CAP_DIM, 1))

    v80 = u8.reshape(B_CAPS, NUM_CAPS * CAP_DIM)
    hc_m = hc_w.transpose(0, 2, 1, 3).reshape(NUM_CAPS, CAP_DIM,
                                              NUM_TARGETS * CAP_DIM)
    out80 = _k4(v80, ssq, hc_m)
    return out80.reshape(1, NUM_TARGETS, CAP_DIM)
